# baseline (speedup 1.0000x reference)
import functools

import jax
import jax.numpy as jnp
from jax import lax
from jax.experimental import pallas as pl
from jax.experimental.pallas import tpu as pltpu
from jax.experimental.pallas import tpu_sc as plsc

NC, NS, LANES = 2, 16, 16
NW = NC * NS

B, L = 4096, 50
D, R = 32, 64
BPW = B // NW
NBUF = 16
GRP = 4

V = 1000001
VPHYS = 1000064
PIECE = 512
NPIECE = 1954
LAST_CS = VPHYS - PIECE
TPW = (NPIECE + NW - 1) // NW
NLINES = VPHYS // 4


def _piece_start(p):
    cs = jnp.where(jnp.int32(p) == NPIECE - 1, LAST_CS, jnp.int32(p) * PIECE)
    return pl.multiple_of(cs, 128)


def _xpose_piece(in_ref, out_ref):
    iota16 = jax.lax.broadcasted_iota(jnp.int32, (LANES,), 0)
    rots = [(iota16 + s) & 15 for s in range(16)]
    rotqs = [r >> 2 for r in rots]
    lanes = {(cb, s): ((rots[s] & 3) << 5) + cb + iota16
             for cb in (0, 16) for s in range(16)}

    def inner(ub16, carry):
        ub = ub16 * 16
        ub4 = ub16 * 4
        for cb in (0, 16):
            c_vec = cb + iota16
            for s0 in (0, 8):
                diags, lines = [], []
                for s in range(s0, s0 + 8):
                    diags.append(
                        plsc.load_gather(in_ref, [c_vec, ub + rots[s]]))
                    lines.append(ub4 + rotqs[s])
                for k, s in enumerate(range(s0, s0 + 8)):
                    plsc.store_scatter(
                        out_ref, [lines[k], lanes[(cb, s)]], diags[k])
        return carry

    lax.fori_loop(0, PIECE // 16, inner, 0)


def _xpose_body(t32_hbm, tp_hbm, in0, in1, out0, out1, sin0, sin1, sout0, sout1):
    wid = lax.axis_index("s") * NC + lax.axis_index("c")
    pltpu.async_copy(t32_hbm.at[:, pl.ds(_piece_start(wid), PIECE)], in0, sin0)
    pltpu.async_copy(t32_hbm.at[:, pl.ds(_piece_start(wid + NW), PIECE)], in1,
                     sin1)

    def step(k, carry):
        for q, in_q, out_q, sin_q, sout_q in (
            (0, in0, out0, sin0, sout0),
            (1, in1, out1, sin1, sout1),
        ):
            t = k * 2 + q
            p = wid + t * NW

            @pl.when(p < NPIECE)
            def _():
                cs = _piece_start(p)
                pltpu.make_async_copy(
                    t32_hbm.at[:, pl.ds(cs, PIECE)], in_q, sin_q).wait()

                @pl.when(t >= 2)
                def _():
                    pcs = _piece_start(p - 2 * NW)
                    pltpu.make_async_copy(
                        out_q,
                        tp_hbm.at[pl.ds(pl.multiple_of(pcs >> 2, 32),
                                        PIECE // 4)],
                        sout_q).wait()

                _xpose_piece(in_q, out_q)
                pltpu.async_copy(
                    out_q,
                    tp_hbm.at[pl.ds(pl.multiple_of(cs >> 2, 32), PIECE // 4)],
                    sout_q)

                @pl.when(p + 2 * NW < NPIECE)
                def _():
                    ncs = _piece_start(p + 2 * NW)
                    pltpu.async_copy(
                        t32_hbm.at[:, pl.ds(ncs, PIECE)], in_q, sin_q)
        return carry

    lax.fori_loop(0, TPW // 2, step, 0)
    pltpu.make_async_copy(out0, tp_hbm.at[pl.ds(0, PIECE // 4)], sout0).wait()
    pltpu.make_async_copy(out1, tp_hbm.at[pl.ds(0, PIECE // 4)], sout1).wait()


_xpose = functools.partial(
    pl.kernel,
    out_type=jax.ShapeDtypeStruct((NLINES, 128), jnp.float32),
    mesh=plsc.VectorSubcoreMesh(core_axis_name="c", subcore_axis_name="s"),
    scratch_types=[
        pltpu.VMEM((D, PIECE), jnp.float32),
        pltpu.VMEM((D, PIECE), jnp.float32),
        pltpu.VMEM((PIECE // 4, 128), jnp.float32),
        pltpu.VMEM((PIECE // 4, 128), jnp.float32),
        pltpu.SemaphoreType.DMA,
        pltpu.SemaphoreType.DMA,
        pltpu.SemaphoreType.DMA,
        pltpu.SemaphoreType.DMA,
    ],
    compiler_params=pltpu.CompilerParams(needs_layout_passes=False),
)(_xpose_body)


_DNUMS = jax.lax.GatherDimensionNumbers(
    offset_dims=(), collapsed_slice_dims=(0,), start_index_map=(0,)
)


def _bcast_lane(vec, d):
    idx = jnp.full((LANES, 1), d, jnp.int32)
    return jax.lax.gather(
        vec, idx, _DNUMS, (1,),
        mode=jax.lax.GatherScatterMode.PROMISE_IN_BOUNDS,
    )


def _lookup_body(inputs_hbm, table_hbm, w_hbm, b_hbm, out_hbm,
                 idx_v, rows_v, w_v, b_v, out_v, sems):
    wid = lax.axis_index("s") * NC + lax.axis_index("c")
    base = wid * BPW
    pltpu.sync_copy(inputs_hbm.at[pl.ds(base, BPW)], idx_v)
    pltpu.sync_copy(w_hbm, w_v)
    pltpu.sync_copy(b_hbm, b_v)

    for j in range(NBUF):
        pltpu.async_copy(table_hbm.at[idx_v.at[j]], rows_v.at[j], sems.at[j])

    def group(g, carry):
        for u in range(GRP):
            i = g * GRP + u
            j = i % NBUF
            pltpu.make_async_copy(
                table_hbm.at[idx_v.at[i]], rows_v.at[j], sems.at[j]
            ).wait()
            buf = rows_v.at[j]
            acc0 = buf[0, 0:16]
            acc1 = buf[0, 16:32]
            for l in range(1, L):
                acc0 = acc0 + buf[l, 0:16]
                acc1 = acc1 + buf[l, 16:32]

            @pl.when(i < BPW - NBUF)
            def _():
                pltpu.async_copy(
                    table_hbm.at[idx_v.at[i + NBUF]], rows_v.at[j], sems.at[j]
                )

            o0 = b_v[0:16]
            o1 = b_v[16:32]
            o2 = b_v[32:48]
            o3 = b_v[48:64]
            for d in range(D):
                s_d = _bcast_lane(acc0 if d < LANES else acc1, d % LANES)
                o0 = o0 + s_d * w_v[d, 0:16]
                o1 = o1 + s_d * w_v[d, 16:32]
                o2 = o2 + s_d * w_v[d, 32:48]
                o3 = o3 + s_d * w_v[d, 48:64]
            out_v[i, 0:16] = o0
            out_v[i, 16:32] = o1
            out_v[i, 32:48] = o2
            out_v[i, 48:64] = o3
        return carry

    lax.fori_loop(0, BPW // GRP, group, 0)
    pltpu.sync_copy(out_v, out_hbm.at[pl.ds(base, BPW)])


_lookup = functools.partial(
    pl.kernel,
    out_type=jax.ShapeDtypeStruct((B, R), jnp.float32),
    mesh=plsc.VectorSubcoreMesh(core_axis_name="c", subcore_axis_name="s"),
    scratch_types=[
        pltpu.VMEM((BPW, L), jnp.int32),
        pltpu.VMEM((NBUF, L, D), jnp.float32),
        pltpu.VMEM((D, R), jnp.float32),
        pltpu.VMEM((R,), jnp.float32),
        pltpu.VMEM((BPW, R), jnp.float32),
        pltpu.SemaphoreType.DMA((NBUF,)),
    ],
    compiler_params=pltpu.CompilerParams(use_tc_tiling_on_sc=False),
)(_lookup_body)


def kernel(inputs, emb_table, W, b):
    t32 = emb_table.T
    tp = _xpose(t32)
    tpv = tp.reshape(VPHYS, D)
    return _lookup(inputs, tpv, W, b)


# speedup vs baseline: 1.0033x; 1.0033x over previous
import functools

import jax
import jax.numpy as jnp
from jax import lax
from jax.experimental import pallas as pl
from jax.experimental.pallas import tpu as pltpu
from jax.experimental.pallas import tpu_sc as plsc

NC, NS, LANES = 2, 16, 16
NW = NC * NS

B, L = 4096, 50
D, R = 32, 64
BPW = B // NW
NBUF = 8
GRP = 4

V = 1000001
VPHYS = 1000064
PIECE = 512
NPIECE = 1954
LAST_CS = VPHYS - PIECE
TPW = (NPIECE + NW - 1) // NW
NLINES = VPHYS // 4


def _piece_start(p):
    cs = jnp.where(jnp.int32(p) == NPIECE - 1, LAST_CS, jnp.int32(p) * PIECE)
    return pl.multiple_of(cs, 128)


def _xpose_piece(in_ref, out_ref):
    iota16 = jax.lax.broadcasted_iota(jnp.int32, (LANES,), 0)
    rots = [(iota16 + s) & 15 for s in range(16)]
    rotqs = [r >> 2 for r in rots]
    lanes = {(cb, s): ((rots[s] & 3) << 5) + cb + iota16
             for cb in (0, 16) for s in range(16)}

    def inner(ub16, carry):
        ub = ub16 * 16
        ub4 = ub16 * 4
        for cb in (0, 16):
            c_vec = cb + iota16
            for s0 in (0, 8):
                diags, lines = [], []
                for s in range(s0, s0 + 8):
                    diags.append(
                        plsc.load_gather(in_ref, [c_vec, ub + rots[s]]))
                    lines.append(ub4 + rotqs[s])
                for k, s in enumerate(range(s0, s0 + 8)):
                    plsc.store_scatter(
                        out_ref, [lines[k], lanes[(cb, s)]], diags[k])
        return carry

    lax.fori_loop(0, PIECE // 16, inner, 0)


def _xpose_body(t32_hbm, tp_hbm, in0, in1, out0, out1, sin0, sin1, sout0, sout1):
    wid = lax.axis_index("s") * NC + lax.axis_index("c")
    pltpu.async_copy(t32_hbm.at[:, pl.ds(_piece_start(wid), PIECE)], in0, sin0)
    pltpu.async_copy(t32_hbm.at[:, pl.ds(_piece_start(wid + NW), PIECE)], in1,
                     sin1)

    def step(k, carry):
        for q, in_q, out_q, sin_q, sout_q in (
            (0, in0, out0, sin0, sout0),
            (1, in1, out1, sin1, sout1),
        ):
            t = k * 2 + q
            p = wid + t * NW

            @pl.when(p < NPIECE)
            def _():
                cs = _piece_start(p)
                pltpu.make_async_copy(
                    t32_hbm.at[:, pl.ds(cs, PIECE)], in_q, sin_q).wait()

                @pl.when(t >= 2)
                def _():
                    pcs = _piece_start(p - 2 * NW)
                    pltpu.make_async_copy(
                        out_q,
                        tp_hbm.at[pl.ds(pl.multiple_of(pcs >> 2, 32),
                                        PIECE // 4)],
                        sout_q).wait()

                _xpose_piece(in_q, out_q)
                pltpu.async_copy(
                    out_q,
                    tp_hbm.at[pl.ds(pl.multiple_of(cs >> 2, 32), PIECE // 4)],
                    sout_q)

                @pl.when(p + 2 * NW < NPIECE)
                def _():
                    ncs = _piece_start(p + 2 * NW)
                    pltpu.async_copy(
                        t32_hbm.at[:, pl.ds(ncs, PIECE)], in_q, sin_q)
        return carry

    lax.fori_loop(0, TPW // 2, step, 0)
    pltpu.make_async_copy(out0, tp_hbm.at[pl.ds(0, PIECE // 4)], sout0).wait()
    pltpu.make_async_copy(out1, tp_hbm.at[pl.ds(0, PIECE // 4)], sout1).wait()


_xpose = functools.partial(
    pl.kernel,
    out_type=jax.ShapeDtypeStruct((NLINES, 128), jnp.float32),
    mesh=plsc.VectorSubcoreMesh(core_axis_name="c", subcore_axis_name="s"),
    scratch_types=[
        pltpu.VMEM((D, PIECE), jnp.float32),
        pltpu.VMEM((D, PIECE), jnp.float32),
        pltpu.VMEM((PIECE // 4, 128), jnp.float32),
        pltpu.VMEM((PIECE // 4, 128), jnp.float32),
        pltpu.SemaphoreType.DMA,
        pltpu.SemaphoreType.DMA,
        pltpu.SemaphoreType.DMA,
        pltpu.SemaphoreType.DMA,
    ],
    compiler_params=pltpu.CompilerParams(needs_layout_passes=False),
)(_xpose_body)


_DNUMS = jax.lax.GatherDimensionNumbers(
    offset_dims=(), collapsed_slice_dims=(0,), start_index_map=(0,)
)


def _bcast_lane(vec, d):
    idx = jnp.full((LANES, 1), d, jnp.int32)
    return jax.lax.gather(
        vec, idx, _DNUMS, (1,),
        mode=jax.lax.GatherScatterMode.PROMISE_IN_BOUNDS,
    )


def _lookup_body(inputs_hbm, table_hbm, w_hbm, b_hbm, out_hbm,
                 idx_v, rows_v, w_v, b_v, out_v, sems):
    wid = lax.axis_index("s") * NC + lax.axis_index("c")
    base = wid * BPW
    pltpu.sync_copy(inputs_hbm.at[pl.ds(base, BPW)], idx_v)
    pltpu.sync_copy(w_hbm, w_v)
    pltpu.sync_copy(b_hbm, b_v)

    for j in range(NBUF):
        pltpu.async_copy(table_hbm.at[idx_v.at[j]], rows_v.at[j], sems.at[j])

    def group(g, carry):
        for u in range(GRP):
            i = g * GRP + u
            j = i % NBUF
            pltpu.make_async_copy(
                table_hbm.at[idx_v.at[i]], rows_v.at[j], sems.at[j]
            ).wait()
            buf = rows_v.at[j]
            acc0 = buf[0, 0:16]
            acc1 = buf[0, 16:32]
            for l in range(1, L):
                acc0 = acc0 + buf[l, 0:16]
                acc1 = acc1 + buf[l, 16:32]

            @pl.when(i < BPW - NBUF)
            def _():
                pltpu.async_copy(
                    table_hbm.at[idx_v.at[i + NBUF]], rows_v.at[j], sems.at[j]
                )

            o0 = b_v[0:16]
            o1 = b_v[16:32]
            o2 = b_v[32:48]
            o3 = b_v[48:64]
            for d in range(D):
                s_d = _bcast_lane(acc0 if d < LANES else acc1, d % LANES)
                o0 = o0 + s_d * w_v[d, 0:16]
                o1 = o1 + s_d * w_v[d, 16:32]
                o2 = o2 + s_d * w_v[d, 32:48]
                o3 = o3 + s_d * w_v[d, 48:64]
            out_v[i, 0:16] = o0
            out_v[i, 16:32] = o1
            out_v[i, 32:48] = o2
            out_v[i, 48:64] = o3
        return carry

    lax.fori_loop(0, BPW // GRP, group, 0)
    pltpu.sync_copy(out_v, out_hbm.at[pl.ds(base, BPW)])


_lookup = functools.partial(
    pl.kernel,
    out_type=jax.ShapeDtypeStruct((B, R), jnp.float32),
    mesh=plsc.VectorSubcoreMesh(core_axis_name="c", subcore_axis_name="s"),
    scratch_types=[
        pltpu.VMEM((BPW, L), jnp.int32),
        pltpu.VMEM((NBUF, L, D), jnp.float32),
        pltpu.VMEM((D, R), jnp.float32),
        pltpu.VMEM((R,), jnp.float32),
        pltpu.VMEM((BPW, R), jnp.float32),
        pltpu.SemaphoreType.DMA((NBUF,)),
    ],
    compiler_params=pltpu.CompilerParams(use_tc_tiling_on_sc=False),
)(_lookup_body)


def kernel(inputs, emb_table, W, b):
    t32 = emb_table.T
    tp = _xpose(t32)
    tpv = tp.reshape(VPHYS, D)
    return _lookup(inputs, tpv, W, b)


# speedup vs baseline: 1.0914x; 1.0877x over previous
import functools

import jax
import jax.numpy as jnp
from jax import lax
from jax.experimental import pallas as pl
from jax.experimental.pallas import tpu as pltpu
from jax.experimental.pallas import tpu_sc as plsc

NC, NS, LANES = 2, 16, 16
NW = NC * NS

B, L = 4096, 50
D, R = 32, 64
BPW = B // NW
NBUF = 8
GRP = 4

V = 1000001
VPHYS = 1000064
PIECE = 512
NPIECE = 1954
LAST_CS = VPHYS - PIECE
TPW = (NPIECE + NW - 1) // NW
NLINES = VPHYS // 4


def _piece_start(p):
    cs = jnp.where(jnp.int32(p) == NPIECE - 1, LAST_CS, jnp.int32(p) * PIECE)
    return pl.multiple_of(cs, 128)


def _xpose_piece(in_ref, out_ref):
    iota16 = jax.lax.broadcasted_iota(jnp.int32, (LANES,), 0)
    rots = [(iota16 + s) & 15 for s in range(16)]
    rotqs = [r >> 2 for r in rots]
    lanes = {(cb, s): ((rots[s] & 3) << 5) + cb + iota16
             for cb in (0, 16) for s in range(16)}

    def inner(ub16, carry):
        ub = ub16 * 16
        ub4 = ub16 * 4
        for cb in (0, 16):
            c_vec = cb + iota16
            for s0 in (0, 8):
                diags, lines = [], []
                for s in range(s0, s0 + 8):
                    diags.append(
                        plsc.load_gather(in_ref, [c_vec, ub + rots[s]]))
                    lines.append(ub4 + rotqs[s])
                for k, s in enumerate(range(s0, s0 + 8)):
                    plsc.store_scatter(
                        out_ref, [lines[k], lanes[(cb, s)]], diags[k])
        return carry

    lax.fori_loop(0, PIECE // 16, inner, 0)


def _xpose_body(t32_hbm, tp_hbm, in0, in1, in2, out0, out1, out2,
                sin0, sin1, sin2, sout0, sout1, sout2):
    wid = lax.axis_index("s") * NC + lax.axis_index("c")
    for j, in_j, sin_j in ((0, in0, sin0), (1, in1, sin1), (2, in2, sin2)):
        pltpu.async_copy(
            t32_hbm.at[:, pl.ds(_piece_start(wid + j * NW), PIECE)],
            in_j, sin_j)

    def step(k, carry):
        for q, in_q, out_q, sin_q, sout_q in (
            (0, in0, out0, sin0, sout0),
            (1, in1, out1, sin1, sout1),
            (2, in2, out2, sin2, sout2),
        ):
            t = k * 3 + q
            p = wid + t * NW

            @pl.when(p < NPIECE)
            def _():
                cs = _piece_start(p)
                pltpu.make_async_copy(
                    t32_hbm.at[:, pl.ds(cs, PIECE)], in_q, sin_q).wait()

                @pl.when(t >= 3)
                def _():
                    pcs = _piece_start(p - 3 * NW)
                    pltpu.make_async_copy(
                        out_q,
                        tp_hbm.at[pl.ds(pl.multiple_of(pcs >> 2, 32),
                                        PIECE // 4)],
                        sout_q).wait()

                _xpose_piece(in_q, out_q)
                pltpu.async_copy(
                    out_q,
                    tp_hbm.at[pl.ds(pl.multiple_of(cs >> 2, 32), PIECE // 4)],
                    sout_q)

                @pl.when(p + 3 * NW < NPIECE)
                def _():
                    ncs = _piece_start(p + 3 * NW)
                    pltpu.async_copy(
                        t32_hbm.at[:, pl.ds(ncs, PIECE)], in_q, sin_q)
        return carry

    lax.fori_loop(0, (TPW + 2) // 3, step, 0)
    pltpu.make_async_copy(out0, tp_hbm.at[pl.ds(0, PIECE // 4)], sout0).wait()
    pltpu.make_async_copy(out1, tp_hbm.at[pl.ds(0, PIECE // 4)], sout1).wait()
    pltpu.make_async_copy(out2, tp_hbm.at[pl.ds(0, PIECE // 4)], sout2).wait()


_xpose = functools.partial(
    pl.kernel,
    out_type=jax.ShapeDtypeStruct((NLINES, 128), jnp.float32),
    mesh=plsc.VectorSubcoreMesh(core_axis_name="c", subcore_axis_name="s"),
    scratch_types=[
        pltpu.VMEM((D, PIECE), jnp.float32),
        pltpu.VMEM((D, PIECE), jnp.float32),
        pltpu.VMEM((D, PIECE), jnp.float32),
        pltpu.VMEM((PIECE // 4, 128), jnp.float32),
        pltpu.VMEM((PIECE // 4, 128), jnp.float32),
        pltpu.VMEM((PIECE // 4, 128), jnp.float32),
        pltpu.SemaphoreType.DMA,
        pltpu.SemaphoreType.DMA,
        pltpu.SemaphoreType.DMA,
        pltpu.SemaphoreType.DMA,
        pltpu.SemaphoreType.DMA,
        pltpu.SemaphoreType.DMA,
    ],
    compiler_params=pltpu.CompilerParams(needs_layout_passes=False),
)(_xpose_body)


_DNUMS = jax.lax.GatherDimensionNumbers(
    offset_dims=(), collapsed_slice_dims=(0,), start_index_map=(0,)
)


def _bcast_lane(vec, d):
    idx = jnp.full((LANES, 1), d, jnp.int32)
    return jax.lax.gather(
        vec, idx, _DNUMS, (1,),
        mode=jax.lax.GatherScatterMode.PROMISE_IN_BOUNDS,
    )


def _lookup_body(inputs_hbm, table_hbm, w_hbm, b_hbm, out_hbm,
                 idx_v, rows_v, w_v, b_v, out_v, sems):
    wid = lax.axis_index("s") * NC + lax.axis_index("c")
    base = wid * BPW
    pltpu.sync_copy(inputs_hbm.at[pl.ds(base, BPW)], idx_v)
    pltpu.sync_copy(w_hbm, w_v)
    pltpu.sync_copy(b_hbm, b_v)

    for j in range(NBUF):
        pltpu.async_copy(table_hbm.at[idx_v.at[j]], rows_v.at[j], sems.at[j])

    def group(g, carry):
        for u in range(GRP):
            i = g * GRP + u
            j = i % NBUF
            pltpu.make_async_copy(
                table_hbm.at[idx_v.at[i]], rows_v.at[j], sems.at[j]
            ).wait()
            buf = rows_v.at[j]
            acc0 = buf[0, 0:16]
            acc1 = buf[0, 16:32]
            for l in range(1, L):
                acc0 = acc0 + buf[l, 0:16]
                acc1 = acc1 + buf[l, 16:32]

            @pl.when(i < BPW - NBUF)
            def _():
                pltpu.async_copy(
                    table_hbm.at[idx_v.at[i + NBUF]], rows_v.at[j], sems.at[j]
                )

            o0 = b_v[0:16]
            o1 = b_v[16:32]
            o2 = b_v[32:48]
            o3 = b_v[48:64]
            for d in range(D):
                s_d = _bcast_lane(acc0 if d < LANES else acc1, d % LANES)
                o0 = o0 + s_d * w_v[d, 0:16]
                o1 = o1 + s_d * w_v[d, 16:32]
                o2 = o2 + s_d * w_v[d, 32:48]
                o3 = o3 + s_d * w_v[d, 48:64]
            out_v[i, 0:16] = o0
            out_v[i, 16:32] = o1
            out_v[i, 32:48] = o2
            out_v[i, 48:64] = o3
        return carry

    lax.fori_loop(0, BPW // GRP, group, 0)
    pltpu.sync_copy(out_v, out_hbm.at[pl.ds(base, BPW)])


_lookup = functools.partial(
    pl.kernel,
    out_type=jax.ShapeDtypeStruct((B, R), jnp.float32),
    mesh=plsc.VectorSubcoreMesh(core_axis_name="c", subcore_axis_name="s"),
    scratch_types=[
        pltpu.VMEM((BPW, L), jnp.int32),
        pltpu.VMEM((NBUF, L, D), jnp.float32),
        pltpu.VMEM((D, R), jnp.float32),
        pltpu.VMEM((R,), jnp.float32),
        pltpu.VMEM((BPW, R), jnp.float32),
        pltpu.SemaphoreType.DMA((NBUF,)),
    ],
    compiler_params=pltpu.CompilerParams(use_tc_tiling_on_sc=False),
)(_lookup_body)


def kernel(inputs, emb_table, W, b):
    t32 = emb_table.T
    tp = _xpose(t32)
    tpv = tp.reshape(VPHYS, D)
    return _lookup(inputs, tpv, W, b)
